# baseline (speedup 1.0000x reference)
import functools

import jax
import jax.numpy as jnp
from jax.experimental import pallas as pl
from jax.experimental.pallas import tpu as pltpu

_ALPHA = 0.2
_WINDOW_SIZES = (1, 2, 3)


def _body(x_ref, t_ref, w_ref, a_ref, o_ref, *, bb, t):
    n = t * bb
    tp = t + 1
    nw = len(_WINDOW_SIZES)
    x2 = x_ref[...].reshape(n, x_ref.shape[-1])
    y = jnp.dot(x2, w_ref[...], preferred_element_type=jnp.float32)
    g = jnp.dot(x2, a_ref[...], preferred_element_type=jnp.float32)
    gT = jnp.transpose(g)

    turns = t_ref[...]
    lane = jax.lax.broadcasted_iota(jnp.int32, (bb, t), 1)
    l2 = jax.lax.broadcasted_iota(jnp.int32, (bb, n), 1)
    s2 = jax.lax.broadcasted_iota(jnp.int32, (bb, n), 0)
    Pm = (l2 % bb) == s2
    trowl = l2 // bb
    l3 = jax.lax.broadcasted_iota(jnp.int32, (n, bb), 1)
    s3 = jax.lax.broadcasted_iota(jnp.int32, (n, bb), 0)
    PT = ((s3 % bb) == l3).astype(jnp.float32)
    l4 = jax.lax.broadcasted_iota(jnp.int32, (n, t), 1)
    s4 = jax.lax.broadcasted_iota(jnp.int32, (n, t), 0)
    tind = (l4 == s4 // bb).astype(jnp.float32)

    A2 = jnp.where(Pm, jnp.broadcast_to(gT[1:2, :], (bb, n)), 0.0)
    g2s = jnp.dot(A2, tind, preferred_element_type=jnp.float32)

    g1r = gT[0:1, :]
    g1rows = [g1r]
    for k in range(1, nw):
        kb = k * bb
        g1rows.append(jnp.concatenate([g1r[:, kb:], g1r[:, :kb]], axis=1))
    g1cols = jnp.transpose(jnp.concatenate(g1rows, axis=0))
    L = jnp.concatenate([PT, g1cols], axis=1)

    rs_l = jax.lax.broadcasted_iota(jnp.int32, (tp, bb), 0)
    ones_rs = (rs_l < t).astype(jnp.float32)

    omega = jnp.zeros((bb, t), jnp.float32)
    nws = jnp.ones((bb, 1), jnp.float32)
    for ws in _WINDOW_SIZES:
        cnt = jnp.maximum(turns - (ws - 2), 0)
        if ws > 1:
            nws = nws + (cnt > 0).astype(jnp.float32)
        f2 = g2s
        for k in range(1, ws):
            f2 = f2 + jnp.concatenate([g2s[:, k:], g2s[:, :k]], axis=1)
        f2 = f2 * (1.0 / ws)
        f2m = jnp.where(lane < cnt, f2, -1e30)
        m2 = jnp.max(f2m, axis=1, keepdims=True)

        top = jnp.concatenate([f2m, m2], axis=1)
        wr_s = jax.lax.broadcasted_iota(jnp.int32, (nw, tp), 0)
        wrows = jnp.where(wr_s < ws, 1.0 / ws, 0.0)
        R = jnp.concatenate([top, wrows], axis=0)

        zf = jnp.dot(L, R, preferred_element_type=jnp.float32)
        zmax = zf[:, t:tp]
        shift = jnp.maximum(zmax, _ALPHA * zmax)
        lr = jnp.maximum(zf, _ALPHA * zf)
        s = jnp.exp(lr - shift)

        r8 = jnp.dot(s, ones_rs,
                     preferred_element_type=jnp.float32)
        rT = jnp.transpose(r8)
        denom = rT * (cnt.astype(jnp.float32) * float(ws))
        Pr = jnp.where(Pm & (trowl < cnt), 1.0 / denom, 0.0)
        q = jnp.dot(Pr, s, preferred_element_type=jnp.float32)

        qv = q[:, :t]
        qc = qv
        for k in range(1, ws):
            qc = qc + jnp.concatenate([qv[:, t - k:], qv[:, :t - k]], axis=1)
        omega = omega + qc

    omega = omega / nws
    wrow = jnp.dot(tind, jnp.transpose(omega),
                   preferred_element_type=jnp.float32)
    Wm = jnp.transpose(wrow * PT)
    out = jnp.dot(Wm, y, preferred_element_type=jnp.float32)
    o_ref[...] = out


@jax.jit
def kernel(local_semantic_vectors, input_turns, W, a1, a2):
    T, B, D = local_semantic_vectors.shape
    BB = 64

    a12 = jnp.concatenate([a1, a2], axis=1)
    wa = jnp.pad(jnp.dot(W, a12), ((0, 0), (0, 6)))
    turns2 = input_turns.astype(jnp.int32).reshape(B, 1)

    body = functools.partial(_body, bb=BB, t=T)

    out = pl.pallas_call(
        body,
        grid=(B // BB,),
        in_specs=[
            pl.BlockSpec((T, BB, D), lambda i: (0, i, 0)),
            pl.BlockSpec((BB, 1), lambda i: (i, 0)),
            pl.BlockSpec((D, D), lambda i: (0, 0)),
            pl.BlockSpec((D, 8), lambda i: (0, 0)),
        ],
        out_specs=pl.BlockSpec((BB, D), lambda i: (i, 0)),
        out_shape=jax.ShapeDtypeStruct((B, D), jnp.float32),
        compiler_params=pltpu.CompilerParams(
            dimension_semantics=("parallel",)),
    )(local_semantic_vectors, turns2, W, wa)
    return out
